# speedup vs baseline: 13.7430x; 13.7430x over previous
import functools

import jax
import jax.numpy as jnp
from jax import lax
from jax.experimental import pallas as pl
from jax.experimental.pallas import tpu as pltpu
from jax.experimental.pallas import tpu_sc as plsc

N = 10000
E = 320000
NFEAT = 128
HIDDEN = 128
NCLASS = 16

NC = 2
NS = 16
NW = NC * NS
CHUNK = 128

NP = 10240
ROWS_PER_TILE = NP // NS
EPT = 10112
E_PAD = EPT * NW
NCH = EPT // CHUNK

_mesh = plsc.VectorSubcoreMesh(core_axis_name="c", subcore_axis_name="s")


def _fill_2d(ref, rows, value):
    vec = jnp.full((16,), value, dtype=jnp.float32)

    def body(i, _):
        r = i // 8
        col = (i % 8) * 16
        ref[r, pl.ds(col, 16)] = vec
        return 0

    lax.fori_loop(0, rows * 8, body, 0)


@functools.partial(
    pl.kernel,
    mesh=_mesh,
    out_type=jax.ShapeDtypeStruct((NC * NP,), jnp.float32),
    scratch_types=[
        pltpu.VMEM((CHUNK,), jnp.float32),
        pltpu.VMEM((CHUNK,), jnp.int32),
        pltpu.VMEM((ROWS_PER_TILE,), jnp.float32),
        pltpu.VMEM_SHARED((NP,), jnp.float32),
        pltpu.SemaphoreType.DMA,
    ],
)
def _deg_kernel(dst_hbm, out_hbm, ones_v, idx_v, zero_v, acc_sh, sem):
    c = lax.axis_index("c")
    s = lax.axis_index("s")
    wid = s * NC + c
    one = jnp.full((16,), 1.0, dtype=jnp.float32)
    zero = jnp.zeros((16,), dtype=jnp.float32)

    def fill_ones(i, _):
        ones_v[pl.ds(i * 16, 16)] = one
        return 0

    lax.fori_loop(0, CHUNK // 16, fill_ones, 0)

    def fill_zero(i, _):
        zero_v[pl.ds(i * 16, 16)] = zero
        return 0

    lax.fori_loop(0, ROWS_PER_TILE // 16, fill_zero, 0)
    pltpu.sync_copy(zero_v, acc_sh.at[pl.ds(s * ROWS_PER_TILE, ROWS_PER_TILE)])
    plsc.subcore_barrier()

    base = wid * EPT

    def body(i, _):
        pltpu.sync_copy(dst_hbm.at[pl.ds(base + i * CHUNK, CHUNK)], idx_v)
        pltpu.sync_copy(ones_v, acc_sh.at[idx_v], add=True)
        return 0

    lax.fori_loop(0, NCH, body, 0)
    plsc.subcore_barrier()

    row0 = s * ROWS_PER_TILE
    pltpu.sync_copy(
        acc_sh.at[pl.ds(row0, ROWS_PER_TILE)],
        out_hbm.at[pl.ds(c * NP + row0, ROWS_PER_TILE)],
    )


@functools.partial(
    pl.kernel,
    mesh=_mesh,
    out_type=jax.ShapeDtypeStruct((NC * NP, HIDDEN), jnp.float32),
    scratch_types=[
        pltpu.VMEM((CHUNK,), jnp.int32),
        pltpu.VMEM((CHUNK,), jnp.int32),
        pltpu.VMEM((CHUNK, HIDDEN), jnp.float32),
        pltpu.VMEM_SHARED((NP, HIDDEN), jnp.float32),
        pltpu.SemaphoreType.DMA,
    ],
)
def _edge_kernel(g_hbm, src_hbm, dst_hbm, out_hbm, idxs_v, idxd_v, rows_v,
                 acc_sh, sem):
    c = lax.axis_index("c")
    s = lax.axis_index("s")
    wid = s * NC + c

    _fill_2d(rows_v, CHUNK, 0.0)
    row0 = s * ROWS_PER_TILE

    def zbody(i, _):
        pltpu.sync_copy(rows_v, acc_sh.at[pl.ds(row0 + i * CHUNK, CHUNK), :])
        return 0

    lax.fori_loop(0, ROWS_PER_TILE // CHUNK, zbody, 0)
    plsc.subcore_barrier()

    base = wid * EPT

    def body(i, _):
        off = base + i * CHUNK
        pltpu.sync_copy(src_hbm.at[pl.ds(off, CHUNK)], idxs_v)
        pltpu.sync_copy(dst_hbm.at[pl.ds(off, CHUNK)], idxd_v)
        pltpu.async_copy(g_hbm.at[idxs_v], rows_v, sem).wait()
        pltpu.sync_copy(rows_v, acc_sh.at[idxd_v], add=True)
        return 0

    lax.fori_loop(0, NCH, body, 0)
    plsc.subcore_barrier()

    def obody(i, _):
        r = row0 + i * CHUNK
        pltpu.sync_copy(acc_sh.at[pl.ds(r, CHUNK), :],
                        out_hbm.at[pl.ds(c * NP + r, CHUNK), :])
        return 0

    lax.fori_loop(0, ROWS_PER_TILE // CHUNK, obody, 0)


BR = 640


def _dense1_body(cnt_ref, x_ref, w1_ref, g_ref, dinv_ref):
    deg = cnt_ref[0] + cnt_ref[1] + 1.0
    dinv = lax.rsqrt(deg)
    h = jnp.dot(x_ref[...], w1_ref[...], preferred_element_type=jnp.float32)
    g_ref[...] = h * dinv
    dinv_ref[...] = dinv


def _dense1(cnt, x_pad, W1):
    return pl.pallas_call(
        _dense1_body,
        grid=(NP // BR,),
        in_specs=[
            pl.BlockSpec((2, BR, 1), lambda i: (0, i, 0)),
            pl.BlockSpec((BR, NFEAT), lambda i: (i, 0)),
            pl.BlockSpec((NFEAT, HIDDEN), lambda i: (0, 0)),
        ],
        out_specs=[
            pl.BlockSpec((BR, HIDDEN), lambda i: (i, 0)),
            pl.BlockSpec((BR, 1), lambda i: (i, 0)),
        ],
        out_shape=[
            jax.ShapeDtypeStruct((NP, HIDDEN), jnp.float32),
            jax.ShapeDtypeStruct((NP, 1), jnp.float32),
        ],
    )(cnt, x_pad, W1)


def _dense2_body(p_ref, g_ref, dinv_ref, b1_ref, w2_ref, b2_ref, o_ref):
    agg = (p_ref[0] + p_ref[1] + g_ref[...]) * dinv_ref[...]
    h1 = jnp.maximum(agg + b1_ref[...], 0.0)
    o = jnp.dot(h1, w2_ref[...], preferred_element_type=jnp.float32)
    o_ref[...] = jnp.maximum(o + b2_ref[...], 0.0)


def _dense2(p, g, dinv, b1, W2p, b2p):
    return pl.pallas_call(
        _dense2_body,
        grid=(NP // BR,),
        in_specs=[
            pl.BlockSpec((2, BR, HIDDEN), lambda i: (0, i, 0)),
            pl.BlockSpec((BR, HIDDEN), lambda i: (i, 0)),
            pl.BlockSpec((BR, 1), lambda i: (i, 0)),
            pl.BlockSpec((1, HIDDEN), lambda i: (0, 0)),
            pl.BlockSpec((HIDDEN, HIDDEN), lambda i: (0, 0)),
            pl.BlockSpec((1, HIDDEN), lambda i: (0, 0)),
        ],
        out_specs=pl.BlockSpec((BR, HIDDEN), lambda i: (i, 0)),
        out_shape=jax.ShapeDtypeStruct((NP, HIDDEN), jnp.float32),
    )(p, g, dinv, b1, W2p, b2p)


@jax.jit
def kernel(x, edge_index, W1, b1, W2, b2):
    src = edge_index[0]
    dst = edge_index[1]
    pad = jnp.full((E_PAD - E,), N, dtype=jnp.int32)
    src_pad = jnp.concatenate([src, pad])
    dst_pad = jnp.concatenate([dst, pad])
    x_pad = jnp.pad(x, ((0, NP - N), (0, 0)))

    cnt = _deg_kernel(dst_pad).reshape(NC, NP, 1)
    g, dinv = _dense1(cnt, x_pad, W1)
    p = _edge_kernel(g, src_pad, dst_pad).reshape(NC, NP, HIDDEN)

    b1r = b1.reshape(1, HIDDEN)
    W2p = jnp.pad(W2, ((0, 0), (0, HIDDEN - NCLASS)))
    b2p = jnp.pad(b2, (0, HIDDEN - NCLASS)).reshape(1, HIDDEN)
    out = _dense2(p, g, dinv, b1r, W2p, b2p)
    return out[:N, :NCLASS]
